# baseline (speedup 1.0000x reference)
import functools

import jax
import jax.numpy as jnp
from jax import lax
from jax.experimental import pallas as pl
from jax.experimental.pallas import tpu as pltpu
from jax.experimental.pallas import tpu_sc as plsc

N = 4096
D_FEAT = 256
EMB = 128
NUM_CLASS = 40
BASE_ROWS = 100000
TOPK = 8
RETRIEVE_W = 0.3
LABEL_W = 0.3

_RB = 256
_NB = N // _RB

_NC, _NS = 2, 16
_NW = _NC * _NS
_CH = 128
_NFULL = BASE_ROWS // _CH
_NCHUNK = 25
_NEG = -3.0e38


def _gcn_body(f_ref, w_ref, a_ref, g_ref, q_ref, h_scr, p_scr, cs_scr):
    i = pl.program_id(0)

    @pl.when(i == 0)
    def _():
        h_scr[...] = jnp.dot(f_ref[...], w_ref[...],
                             preferred_element_type=jnp.float32)
        cs_scr[...] = jnp.zeros_like(cs_scr)

    ablk = a_ref[...]
    p_scr[pl.ds(i * _RB, _RB), :] = jnp.tanh(
        jnp.dot(ablk, h_scr[...], preferred_element_type=jnp.float32))
    cs_scr[...] += jnp.sum(ablk, axis=0, keepdims=True)

    @pl.when(i == _NB - 1)
    def _():
        p_all = p_scr[...]
        g_ref[...] = jnp.sum(p_all, axis=0, keepdims=True) * (1.0 / N)
        q_ref[...] = jnp.dot(cs_scr[...] * (1.0 / N), p_all,
                             preferred_element_type=jnp.float32)


def _gcn_call(features, adj, W_pre, interpret=False):
    return pl.pallas_call(
        _gcn_body,
        grid=(_NB,),
        in_specs=[
            pl.BlockSpec((N, D_FEAT), lambda i: (0, 0)),
            pl.BlockSpec((D_FEAT, EMB), lambda i: (0, 0)),
            pl.BlockSpec((_RB, N), lambda i: (i, 0)),
        ],
        out_specs=[
            pl.BlockSpec((1, EMB), lambda i: (0, 0)),
            pl.BlockSpec((1, EMB), lambda i: (0, 0)),
        ],
        out_shape=[jax.ShapeDtypeStruct((1, EMB), jnp.float32)] * 2,
        scratch_shapes=[
            pltpu.VMEM((N, EMB), jnp.float32),
            pltpu.VMEM((N, EMB), jnp.float32),
            pltpu.VMEM((1, N), jnp.float32),
        ],
        interpret=interpret,
    )(features, W_pre, adj)


def _merge_top16(tv, ti, cv, ci):
    cvr = lax.rev(cv, (0,))
    cir = lax.rev(ci, (0,))
    keep = tv >= cvr
    mv = jnp.where(keep, tv, cvr)
    mi = jnp.where(keep, ti, cir)
    rv, ri = plsc.sort_key_val(mv, mi, descending=True)
    return rv, ri


def _scan_call(base_emb, g_vec):
    mesh = plsc.VectorSubcoreMesh(core_axis_name="c", subcore_axis_name="s",
                                  num_cores=_NC, num_subcores=_NS)

    @functools.partial(
        pl.kernel,
        out_type=(jax.ShapeDtypeStruct((_NW * 16,), jnp.float32),
                  jax.ShapeDtypeStruct((_NW * 16,), jnp.int32)),
        mesh=mesh,
        scratch_types=[
            pltpu.VMEM((2, _CH, EMB), jnp.float32),
            pltpu.VMEM((EMB,), jnp.float32),
            pltpu.VMEM((EMB * 16,), jnp.float32),
            pltpu.VMEM((16,), jnp.float32),
            pltpu.VMEM((16,), jnp.int32),
            pltpu.SemaphoreType.DMA,
            pltpu.SemaphoreType.DMA,
        ],
        compiler_params=pltpu.CompilerParams(needs_layout_passes=False),
    )
    def scan_k(emb_hbm, g_hbm, keys_out, rows_out,
               ebuf, qbuf, qrot, tvbuf, tibuf, sem0, sem1):
        cid = lax.axis_index("c")
        sid = lax.axis_index("s")
        wid = sid * _NC + cid
        lanes = lax.iota(jnp.int32, 16)

        def chunk_row0(t):
            c_eff = jnp.minimum(wid + _NW * t, _NFULL)
            row0 = jnp.minimum(c_eff * _CH, BASE_ROWS - _CH)
            return pl.multiple_of(row0, 8)

        pltpu.sync_copy(g_hbm, qbuf)

        def build_qrot(c, carry):
            qv = plsc.load_gather(qbuf,
                                  [jnp.bitwise_and(c + lanes, EMB - 1)])
            qrot[pl.ds(c * 16, 16)] = qv
            return carry

        lax.fori_loop(0, EMB, build_qrot, 0)

        sems = (sem0, sem1)

        def dma_start(t, slot):
            pltpu.async_copy(emb_hbm.at[pl.ds(chunk_row0(t), _CH)],
                             ebuf.at[slot], sems[slot])

        def dma_wait(slot):
            pltpu.make_async_copy(emb_hbm.at[pl.ds(0, _CH)],
                                  ebuf.at[slot], sems[slot]).wait()

        def compute_chunk(t, slot, tv, ti):
            c = wid + _NW * t
            in_range = c <= _NFULL
            c_eff = jnp.minimum(c, _NFULL)
            row0 = chunk_row0(t)
            slot_vec = jnp.full((16,), slot, jnp.int32)
            rowvs, valids, ivs = [], [], []
            for gi in range(8):
                roff = gi * 16 + lanes
                grow = row0 + roff
                validv = jnp.logical_and(
                    jnp.logical_and(grow >= c_eff * _CH,
                                    grow < BASE_ROWS),
                    in_range)
                rowvs.append(roff)
                valids.append(validv)
                ivs.append(grow)

            def colpair(cb, carry):
                accs = list(carry[0])
                nacs = list(carry[1])
                for j in range(4):
                    cc = cb * 4 + j
                    cl = jnp.bitwise_and(cc + lanes, EMB - 1)
                    qv = qrot[pl.ds(cc * 16, 16)]
                    for gi in range(8):
                        v = plsc.load_gather(ebuf,
                                             [slot_vec, rowvs[gi], cl])
                        accs[gi] = accs[gi] + v * qv
                        nacs[gi] = nacs[gi] + v * v
                return tuple(accs), tuple(nacs)

            z = tuple(jnp.zeros((16,), jnp.float32) for _ in range(8))
            accs, nacs = lax.fori_loop(0, EMB // 4, colpair, (z, z))
            for gi in range(8):
                key = accs[gi] * jnp.abs(accs[gi]) / nacs[gi]
                key = jnp.where(valids[gi], key, _NEG)
                cv, ci = plsc.sort_key_val(key, ivs[gi], descending=True)
                tv, ti = _merge_top16(tv, ti, cv, ci)
            return tv, ti

        tv = jnp.full((16,), _NEG, jnp.float32)
        ti = jnp.zeros((16,), jnp.int32)
        dma_start(0, 0)
        dma_start(1, 1)

        def pair(c2, c):
            tv, ti = c
            ch0 = 2 * c2
            dma_wait(0)
            tv, ti = compute_chunk(ch0, 0, tv, ti)
            dma_start(ch0 + 2, 0)

            dma_wait(1)
            tv, ti = compute_chunk(ch0 + 1, 1, tv, ti)

            @pl.when(ch0 + 3 < _NCHUNK)
            def _():
                dma_start(ch0 + 3, 1)

            return tv, ti

        tv, ti = lax.fori_loop(0, (_NCHUNK - 1) // 2, pair, (tv, ti))
        dma_wait(0)
        tv, ti = compute_chunk(jnp.int32(_NCHUNK - 1), 0, tv, ti)

        tvbuf[...] = tv
        tibuf[...] = ti
        pltpu.sync_copy(tvbuf, keys_out.at[pl.ds(wid * 16, 16)])
        pltpu.sync_copy(tibuf, rows_out.at[pl.ds(wid * 16, 16)])

    return scan_k(base_emb, g_vec)


def _pick_call(base_emb, keys, rows):
    mesh = plsc.VectorSubcoreMesh(core_axis_name="c", subcore_axis_name="s",
                                  num_cores=_NC, num_subcores=_NS)

    @functools.partial(
        pl.kernel,
        out_type=(jax.ShapeDtypeStruct((EMB,), jnp.float32),
                  jax.ShapeDtypeStruct((16,), jnp.int32)),
        mesh=mesh,
        scratch_types=[
            pltpu.VMEM((_NW * 16,), jnp.float32),
            pltpu.VMEM((_NW * 16,), jnp.int32),
            pltpu.VMEM((16,), jnp.int32),
            pltpu.VMEM((16, EMB), jnp.float32),
            pltpu.VMEM((EMB,), jnp.float32),
            pltpu.SemaphoreType.DMA,
        ],
        compiler_params=pltpu.CompilerParams(needs_layout_passes=False),
    )
    def pick_k(emb_hbm, keys_hbm, rows_hbm, re_out, top8_out,
               kb, ib, tib, eb, oe, sem):
        cid = lax.axis_index("c")
        sid = lax.axis_index("s")

        @pl.when((cid == 0) & (sid == 0))
        def _():
            pltpu.sync_copy(keys_hbm, kb)
            pltpu.sync_copy(rows_hbm, ib)

            def mrg(w, c):
                tv, ti = c
                tv, ti = _merge_top16(tv, ti, kb[pl.ds(w * 16, 16)],
                                      ib[pl.ds(w * 16, 16)])
                return tv, ti

            tv = jnp.full((16,), _NEG, jnp.float32)
            ti = jnp.zeros((16,), jnp.int32)
            tv, ti = lax.fori_loop(0, _NW, mrg, (tv, ti))

            tib[...] = ti
            pltpu.sync_copy(tib, top8_out)
            pltpu.async_copy(emb_hbm.at[tib], eb, sem).wait()

            for k in range(8):
                s = jnp.zeros((16,), jnp.float32)
                for p in range(TOPK):
                    s = s + eb[p, pl.ds(16 * k, 16)]
                oe[pl.ds(16 * k, 16)] = s
            pltpu.sync_copy(oe, re_out)

    return pick_k(base_emb, keys, rows)


def _decode_body(idx_ref, lab_ref, q_ref, re_ref, w1_ref, b1_ref, w2_ref,
                 b2_ref, o_ref, acc_scr):
    i = pl.program_id(0)

    @pl.when(i == 0)
    def _():
        acc_scr[...] = jnp.zeros_like(acc_scr)

    col = idx_ref[i]
    b = col // 128
    c_in = col - b * 128
    tile = lab_ref[...]
    li = lax.broadcasted_iota(jnp.int32, (NUM_CLASS, 128), 1)
    tile = jnp.where(li < BASE_ROWS - b * 128, tile, 0.0)
    sel = (lax.broadcasted_iota(jnp.int32, (1, 128), 1)
           == c_in).astype(jnp.float32)
    acc_scr[...] += lax.dot_general(sel, tile, (((1,), (1,)), ((), ())),
                                    preferred_element_type=jnp.float32)

    @pl.when(i == TOPK - 1)
    def _():
        rag_label = acc_scr[...] * (1.0 / TOPK)
        hidden = (q_ref[...] * (1.0 - RETRIEVE_W)
                  + re_ref[...] * RETRIEVE_W)
        h1 = jnp.dot(hidden, w1_ref[...],
                     preferred_element_type=jnp.float32) + b1_ref[...]
        h1 = jnp.maximum(h1, 0.0)
        logits = jnp.dot(h1, w2_ref[...],
                         preferred_element_type=jnp.float32) + b2_ref[...]
        m = jnp.max(logits, axis=1, keepdims=True)
        e = jnp.exp(logits - m)
        sm = e / jnp.sum(e, axis=1, keepdims=True)
        o_ref[...] = sm * (1.0 - LABEL_W) + rag_label * LABEL_W


def _decode_call(top8, labels3d, q, rag_e, W1, b1, W2, b2, interpret=False):
    grid_spec = pltpu.PrefetchScalarGridSpec(
        num_scalar_prefetch=1,
        grid=(TOPK,),
        in_specs=[
            pl.BlockSpec((NUM_CLASS, 128), lambda i, idx: (0, idx[i] // 128)),
            pl.BlockSpec((1, EMB), lambda i, idx: (0, 0)),
            pl.BlockSpec((1, EMB), lambda i, idx: (0, 0)),
            pl.BlockSpec((EMB, EMB), lambda i, idx: (0, 0)),
            pl.BlockSpec((1, EMB), lambda i, idx: (0, 0)),
            pl.BlockSpec((EMB, NUM_CLASS), lambda i, idx: (0, 0)),
            pl.BlockSpec((1, NUM_CLASS), lambda i, idx: (0, 0)),
        ],
        out_specs=pl.BlockSpec((1, NUM_CLASS), lambda i, idx: (0, 0)),
        scratch_shapes=[pltpu.VMEM((1, NUM_CLASS), jnp.float32)],
    )
    return pl.pallas_call(
        _decode_body,
        grid_spec=grid_spec,
        out_shape=jax.ShapeDtypeStruct((1, NUM_CLASS), jnp.float32),
        interpret=interpret,
    )(top8, labels3d, q, rag_e, W1, b1, W2, b2)


def kernel(features, adj, W_pre, base_emb, base_labels, W1, b1, W2, b2):
    g2d, q2d = _gcn_call(features, adj, W_pre)
    keys, rows = _scan_call(base_emb, g2d.reshape((EMB,)))
    rag_e, top8 = _pick_call(base_emb, keys, rows)
    return _decode_call(top8, base_labels.T,
                        q2d, rag_e.reshape((1, EMB)),
                        W1, b1.reshape((1, EMB)), W2,
                        b2.reshape((1, NUM_CLASS)))


# speedup vs baseline: 1.0384x; 1.0384x over previous
import functools

import jax
import jax.numpy as jnp
from jax import lax
from jax.experimental import pallas as pl
from jax.experimental.pallas import tpu as pltpu
from jax.experimental.pallas import tpu_sc as plsc

N = 4096
D_FEAT = 256
EMB = 128
NUM_CLASS = 40
BASE_ROWS = 100000
TOPK = 8
RETRIEVE_W = 0.3
LABEL_W = 0.3

_RB = 512
_NB = N // _RB

_NC, _NS = 2, 16
_NW = _NC * _NS
_CH = 128
_NFULL = BASE_ROWS // _CH
_NCHUNK = 25
_NEG = -3.0e38


def _gcn_body(f_ref, w_ref, a_ref, g_ref, q_ref, h_scr, p_scr, cs_scr):
    i = pl.program_id(0)

    @pl.when(i == 0)
    def _():
        h_scr[...] = jnp.dot(f_ref[...], w_ref[...],
                             preferred_element_type=jnp.float32)
        cs_scr[...] = jnp.zeros_like(cs_scr)

    ablk = a_ref[...]
    p_scr[pl.ds(i * _RB, _RB), :] = jnp.tanh(
        jnp.dot(ablk, h_scr[...], preferred_element_type=jnp.float32))
    cs_scr[...] += jnp.sum(ablk, axis=0, keepdims=True)

    @pl.when(i == _NB - 1)
    def _():
        p_all = p_scr[...]
        g_ref[...] = jnp.sum(p_all, axis=0, keepdims=True) * (1.0 / N)
        q_ref[...] = jnp.dot(cs_scr[...] * (1.0 / N), p_all,
                             preferred_element_type=jnp.float32)


def _gcn_call(features, adj, W_pre, interpret=False):
    return pl.pallas_call(
        _gcn_body,
        grid=(_NB,),
        in_specs=[
            pl.BlockSpec((N, D_FEAT), lambda i: (0, 0)),
            pl.BlockSpec((D_FEAT, EMB), lambda i: (0, 0)),
            pl.BlockSpec((_RB, N), lambda i: (i, 0)),
        ],
        out_specs=[
            pl.BlockSpec((1, EMB), lambda i: (0, 0)),
            pl.BlockSpec((1, EMB), lambda i: (0, 0)),
        ],
        out_shape=[jax.ShapeDtypeStruct((1, EMB), jnp.float32)] * 2,
        scratch_shapes=[
            pltpu.VMEM((N, EMB), jnp.float32),
            pltpu.VMEM((N, EMB), jnp.float32),
            pltpu.VMEM((1, N), jnp.float32),
        ],
        interpret=interpret,
    )(features, W_pre, adj)


def _merge_top16(tv, ti, cv, ci):
    cvr = lax.rev(cv, (0,))
    cir = lax.rev(ci, (0,))
    keep = tv >= cvr
    mv = jnp.where(keep, tv, cvr)
    mi = jnp.where(keep, ti, cir)
    rv, ri = plsc.sort_key_val(mv, mi, descending=True)
    return rv, ri


def _scan_call(base_emb, g_vec):
    mesh = plsc.VectorSubcoreMesh(core_axis_name="c", subcore_axis_name="s",
                                  num_cores=_NC, num_subcores=_NS)

    @functools.partial(
        pl.kernel,
        out_type=(jax.ShapeDtypeStruct((_NW * 16,), jnp.float32),
                  jax.ShapeDtypeStruct((_NW * 16,), jnp.int32)),
        mesh=mesh,
        scratch_types=[
            pltpu.VMEM((2, _CH, EMB), jnp.float32),
            pltpu.VMEM((EMB,), jnp.float32),
            pltpu.VMEM((EMB * 16,), jnp.float32),
            pltpu.VMEM((16,), jnp.float32),
            pltpu.VMEM((16,), jnp.int32),
            pltpu.SemaphoreType.DMA,
            pltpu.SemaphoreType.DMA,
        ],
        compiler_params=pltpu.CompilerParams(needs_layout_passes=False),
    )
    def scan_k(emb_hbm, g_hbm, keys_out, rows_out,
               ebuf, qbuf, qrot, tvbuf, tibuf, sem0, sem1):
        cid = lax.axis_index("c")
        sid = lax.axis_index("s")
        wid = sid * _NC + cid
        lanes = lax.iota(jnp.int32, 16)

        def chunk_row0(t):
            c_eff = jnp.minimum(wid + _NW * t, _NFULL)
            row0 = jnp.minimum(c_eff * _CH, BASE_ROWS - _CH)
            return pl.multiple_of(row0, 8)

        pltpu.sync_copy(g_hbm, qbuf)

        def build_qrot(c, carry):
            qv = plsc.load_gather(qbuf,
                                  [jnp.bitwise_and(c + lanes, EMB - 1)])
            qrot[pl.ds(c * 16, 16)] = qv
            return carry

        lax.fori_loop(0, EMB, build_qrot, 0)

        sems = (sem0, sem1)

        def dma_start(t, slot):
            pltpu.async_copy(emb_hbm.at[pl.ds(chunk_row0(t), _CH)],
                             ebuf.at[slot], sems[slot])

        def dma_wait(slot):
            pltpu.make_async_copy(emb_hbm.at[pl.ds(0, _CH)],
                                  ebuf.at[slot], sems[slot]).wait()

        def compute_chunk(t, slot, tv, ti):
            c = wid + _NW * t
            in_range = c <= _NFULL
            c_eff = jnp.minimum(c, _NFULL)
            row0 = chunk_row0(t)
            slot_vec = jnp.full((16,), slot, jnp.int32)
            rowvs, valids, ivs = [], [], []
            for gi in range(8):
                roff = gi * 16 + lanes
                grow = row0 + roff
                validv = jnp.logical_and(
                    jnp.logical_and(grow >= c_eff * _CH,
                                    grow < BASE_ROWS),
                    in_range)
                rowvs.append(roff)
                valids.append(validv)
                ivs.append(grow)

            def colpair(cb, carry):
                accs = list(carry[0])
                nacs = list(carry[1])
                for j in range(4):
                    cc = cb * 4 + j
                    cl = jnp.bitwise_and(cc + lanes, EMB - 1)
                    qv = qrot[pl.ds(cc * 16, 16)]
                    for gi in range(8):
                        v = plsc.load_gather(ebuf,
                                             [slot_vec, rowvs[gi], cl])
                        accs[gi] = accs[gi] + v * qv
                        nacs[gi] = nacs[gi] + v * v
                return tuple(accs), tuple(nacs)

            z = tuple(jnp.zeros((16,), jnp.float32) for _ in range(8))
            accs, nacs = lax.fori_loop(0, EMB // 4, colpair, (z, z))
            for gi in range(8):
                key = accs[gi] * jnp.abs(accs[gi]) / nacs[gi]
                key = jnp.where(valids[gi], key, _NEG)
                cv, ci = plsc.sort_key_val(key, ivs[gi], descending=True)
                tv, ti = _merge_top16(tv, ti, cv, ci)
            return tv, ti

        tv = jnp.full((16,), _NEG, jnp.float32)
        ti = jnp.zeros((16,), jnp.int32)
        dma_start(0, 0)
        dma_start(1, 1)

        def pair(c2, c):
            tv, ti = c
            ch0 = 2 * c2
            dma_wait(0)
            tv, ti = compute_chunk(ch0, 0, tv, ti)
            dma_start(ch0 + 2, 0)

            dma_wait(1)
            tv, ti = compute_chunk(ch0 + 1, 1, tv, ti)

            @pl.when(ch0 + 3 < _NCHUNK)
            def _():
                dma_start(ch0 + 3, 1)

            return tv, ti

        tv, ti = lax.fori_loop(0, (_NCHUNK - 1) // 2, pair, (tv, ti))
        dma_wait(0)
        tv, ti = compute_chunk(jnp.int32(_NCHUNK - 1), 0, tv, ti)

        tvbuf[...] = tv
        tibuf[...] = ti
        pltpu.sync_copy(tvbuf, keys_out.at[pl.ds(wid * 16, 16)])
        pltpu.sync_copy(tibuf, rows_out.at[pl.ds(wid * 16, 16)])

    return scan_k(base_emb, g_vec)


def _pick_call(base_emb, keys, rows):
    mesh = plsc.VectorSubcoreMesh(core_axis_name="c", subcore_axis_name="s",
                                  num_cores=_NC, num_subcores=_NS)

    @functools.partial(
        pl.kernel,
        out_type=(jax.ShapeDtypeStruct((EMB,), jnp.float32),
                  jax.ShapeDtypeStruct((16,), jnp.int32)),
        mesh=mesh,
        scratch_types=[
            pltpu.VMEM((_NW * 16,), jnp.float32),
            pltpu.VMEM((_NW * 16,), jnp.int32),
            pltpu.VMEM((16,), jnp.int32),
            pltpu.VMEM((16, EMB), jnp.float32),
            pltpu.VMEM((EMB,), jnp.float32),
            pltpu.SemaphoreType.DMA,
        ],
        compiler_params=pltpu.CompilerParams(needs_layout_passes=False),
    )
    def pick_k(emb_hbm, keys_hbm, rows_hbm, re_out, top8_out,
               kb, ib, tib, eb, oe, sem):
        cid = lax.axis_index("c")
        sid = lax.axis_index("s")

        @pl.when((cid == 0) & (sid == 0))
        def _():
            pltpu.sync_copy(keys_hbm, kb)
            pltpu.sync_copy(rows_hbm, ib)

            def mrg(w, c):
                tv, ti = c
                tv, ti = _merge_top16(tv, ti, kb[pl.ds(w * 16, 16)],
                                      ib[pl.ds(w * 16, 16)])
                return tv, ti

            tv = jnp.full((16,), _NEG, jnp.float32)
            ti = jnp.zeros((16,), jnp.int32)
            tv, ti = lax.fori_loop(0, _NW, mrg, (tv, ti))

            tib[...] = ti
            pltpu.sync_copy(tib, top8_out)
            pltpu.async_copy(emb_hbm.at[tib], eb, sem).wait()

            for k in range(8):
                s = jnp.zeros((16,), jnp.float32)
                for p in range(TOPK):
                    s = s + eb[p, pl.ds(16 * k, 16)]
                oe[pl.ds(16 * k, 16)] = s
            pltpu.sync_copy(oe, re_out)

    return pick_k(base_emb, keys, rows)


def _decode_body(idx_ref, lab_ref, q_ref, re_ref, w1_ref, b1_ref, w2_ref,
                 b2_ref, o_ref, acc_scr):
    i = pl.program_id(0)

    @pl.when(i == 0)
    def _():
        acc_scr[...] = jnp.zeros_like(acc_scr)

    col = idx_ref[i]
    b = col // 128
    c_in = col - b * 128
    tile = lab_ref[...]
    li = lax.broadcasted_iota(jnp.int32, (NUM_CLASS, 128), 1)
    tile = jnp.where(li < BASE_ROWS - b * 128, tile, 0.0)
    sel = (lax.broadcasted_iota(jnp.int32, (1, 128), 1)
           == c_in).astype(jnp.float32)
    acc_scr[...] += lax.dot_general(sel, tile, (((1,), (1,)), ((), ())),
                                    preferred_element_type=jnp.float32)

    @pl.when(i == TOPK - 1)
    def _():
        rag_label = acc_scr[...] * (1.0 / TOPK)
        hidden = (q_ref[...] * (1.0 - RETRIEVE_W)
                  + re_ref[...] * RETRIEVE_W)
        h1 = jnp.dot(hidden, w1_ref[...],
                     preferred_element_type=jnp.float32) + b1_ref[...]
        h1 = jnp.maximum(h1, 0.0)
        logits = jnp.dot(h1, w2_ref[...],
                         preferred_element_type=jnp.float32) + b2_ref[...]
        m = jnp.max(logits, axis=1, keepdims=True)
        e = jnp.exp(logits - m)
        sm = e / jnp.sum(e, axis=1, keepdims=True)
        o_ref[...] = sm * (1.0 - LABEL_W) + rag_label * LABEL_W


def _decode_call(top8, labels3d, q, rag_e, W1, b1, W2, b2, interpret=False):
    grid_spec = pltpu.PrefetchScalarGridSpec(
        num_scalar_prefetch=1,
        grid=(TOPK,),
        in_specs=[
            pl.BlockSpec((NUM_CLASS, 128), lambda i, idx: (0, idx[i] // 128)),
            pl.BlockSpec((1, EMB), lambda i, idx: (0, 0)),
            pl.BlockSpec((1, EMB), lambda i, idx: (0, 0)),
            pl.BlockSpec((EMB, EMB), lambda i, idx: (0, 0)),
            pl.BlockSpec((1, EMB), lambda i, idx: (0, 0)),
            pl.BlockSpec((EMB, NUM_CLASS), lambda i, idx: (0, 0)),
            pl.BlockSpec((1, NUM_CLASS), lambda i, idx: (0, 0)),
        ],
        out_specs=pl.BlockSpec((1, NUM_CLASS), lambda i, idx: (0, 0)),
        scratch_shapes=[pltpu.VMEM((1, NUM_CLASS), jnp.float32)],
    )
    return pl.pallas_call(
        _decode_body,
        grid_spec=grid_spec,
        out_shape=jax.ShapeDtypeStruct((1, NUM_CLASS), jnp.float32),
        interpret=interpret,
    )(top8, labels3d, q, rag_e, W1, b1, W2, b2)


def kernel(features, adj, W_pre, base_emb, base_labels, W1, b1, W2, b2):
    g2d, q2d = _gcn_call(features, adj, W_pre)
    keys, rows = _scan_call(base_emb, g2d.reshape((EMB,)))
    rag_e, top8 = _pick_call(base_emb, keys, rows)
    return _decode_call(top8, base_labels.T,
                        q2d, rag_e.reshape((1, EMB)),
                        W1, b1.reshape((1, EMB)), W2,
                        b2.reshape((1, NUM_CLASS)))


# speedup vs baseline: 1.0571x; 1.0179x over previous
import functools

import jax
import jax.numpy as jnp
from jax import lax
from jax.experimental import pallas as pl
from jax.experimental.pallas import tpu as pltpu
from jax.experimental.pallas import tpu_sc as plsc

N = 4096
D_FEAT = 256
EMB = 128
NUM_CLASS = 40
BASE_ROWS = 100000
TOPK = 8
RETRIEVE_W = 0.3
LABEL_W = 0.3

_RB = 512
_NB = N // _RB

_NC, _NS = 2, 16
_NW = _NC * _NS
_CH = 256
_NFULL = BASE_ROWS // _CH
_NCHUNK = 13
_NEG = -3.0e38


def _gcn_body(f_ref, w_ref, a_ref, g_ref, q_ref, h_scr, p_scr, cs_scr):
    i = pl.program_id(0)

    @pl.when(i == 0)
    def _():
        h_scr[...] = jnp.dot(f_ref[...], w_ref[...],
                             preferred_element_type=jnp.float32)
        cs_scr[...] = jnp.zeros_like(cs_scr)

    ablk = a_ref[...]
    p_scr[pl.ds(i * _RB, _RB), :] = jnp.tanh(
        jnp.dot(ablk, h_scr[...], preferred_element_type=jnp.float32))
    cs_scr[...] += jnp.sum(ablk, axis=0, keepdims=True)

    @pl.when(i == _NB - 1)
    def _():
        p_all = p_scr[...]
        g_ref[...] = jnp.sum(p_all, axis=0, keepdims=True) * (1.0 / N)
        q_ref[...] = jnp.dot(cs_scr[...] * (1.0 / N), p_all,
                             preferred_element_type=jnp.float32)


def _gcn_call(features, adj, W_pre, interpret=False):
    return pl.pallas_call(
        _gcn_body,
        grid=(_NB,),
        in_specs=[
            pl.BlockSpec((N, D_FEAT), lambda i: (0, 0)),
            pl.BlockSpec((D_FEAT, EMB), lambda i: (0, 0)),
            pl.BlockSpec((_RB, N), lambda i: (i, 0)),
        ],
        out_specs=[
            pl.BlockSpec((1, EMB), lambda i: (0, 0)),
            pl.BlockSpec((1, EMB), lambda i: (0, 0)),
        ],
        out_shape=[jax.ShapeDtypeStruct((1, EMB), jnp.float32)] * 2,
        scratch_shapes=[
            pltpu.VMEM((N, EMB), jnp.float32),
            pltpu.VMEM((N, EMB), jnp.float32),
            pltpu.VMEM((1, N), jnp.float32),
        ],
        interpret=interpret,
    )(features, W_pre, adj)


def _merge_top16(tv, ti, cv, ci):
    cvr = lax.rev(cv, (0,))
    cir = lax.rev(ci, (0,))
    keep = tv >= cvr
    mv = jnp.where(keep, tv, cvr)
    mi = jnp.where(keep, ti, cir)
    rv, ri = plsc.sort_key_val(mv, mi, descending=True)
    return rv, ri


def _scan_call(base_emb, g_vec):
    mesh = plsc.VectorSubcoreMesh(core_axis_name="c", subcore_axis_name="s",
                                  num_cores=_NC, num_subcores=_NS)

    @functools.partial(
        pl.kernel,
        out_type=(jax.ShapeDtypeStruct((_NW * 16,), jnp.float32),
                  jax.ShapeDtypeStruct((_NW * 16,), jnp.int32)),
        mesh=mesh,
        scratch_types=[
            pltpu.VMEM((2, _CH, EMB), jnp.float32),
            pltpu.VMEM((EMB,), jnp.float32),
            pltpu.VMEM((EMB * 16,), jnp.float32),
            pltpu.VMEM((16,), jnp.float32),
            pltpu.VMEM((16,), jnp.int32),
            pltpu.SemaphoreType.DMA,
            pltpu.SemaphoreType.DMA,
        ],
        compiler_params=pltpu.CompilerParams(needs_layout_passes=False),
    )
    def scan_k(emb_hbm, g_hbm, keys_out, rows_out,
               ebuf, qbuf, qrot, tvbuf, tibuf, sem0, sem1):
        cid = lax.axis_index("c")
        sid = lax.axis_index("s")
        wid = sid * _NC + cid
        lanes = lax.iota(jnp.int32, 16)

        def chunk_row0(t):
            c_eff = jnp.minimum(wid + _NW * t, _NFULL)
            row0 = jnp.minimum(c_eff * _CH, BASE_ROWS - _CH)
            return pl.multiple_of(row0, 8)

        pltpu.sync_copy(g_hbm, qbuf)

        def build_qrot(c, carry):
            qv = plsc.load_gather(qbuf,
                                  [jnp.bitwise_and(c + lanes, EMB - 1)])
            qrot[pl.ds(c * 16, 16)] = qv
            return carry

        lax.fori_loop(0, EMB, build_qrot, 0)

        sems = (sem0, sem1)

        def dma_start(t, slot):
            pltpu.async_copy(emb_hbm.at[pl.ds(chunk_row0(t), _CH)],
                             ebuf.at[slot], sems[slot])

        def dma_wait(slot):
            pltpu.make_async_copy(emb_hbm.at[pl.ds(0, _CH)],
                                  ebuf.at[slot], sems[slot]).wait()

        def compute_chunk(t, slot, tv, ti):
            c = wid + _NW * t
            in_range = c <= _NFULL
            c_eff = jnp.minimum(c, _NFULL)
            row0 = chunk_row0(t)
            slot_vec = jnp.full((16,), slot, jnp.int32)
            for half in range(_CH // 128):
                rowvs, valids, ivs = [], [], []
                for gi in range(8):
                    roff = half * 128 + gi * 16 + lanes
                    grow = row0 + roff
                    validv = jnp.logical_and(
                        jnp.logical_and(grow >= c_eff * _CH,
                                        grow < BASE_ROWS),
                        in_range)
                    rowvs.append(roff)
                    valids.append(validv)
                    ivs.append(grow)

                def colpair(cb, carry):
                    accs = list(carry[0])
                    nacs = list(carry[1])
                    for j in range(2):
                        cc = cb * 2 + j
                        cl = jnp.bitwise_and(cc + lanes, EMB - 1)
                        qv = qrot[pl.ds(cc * 16, 16)]
                        for gi in range(8):
                            v = plsc.load_gather(ebuf,
                                                 [slot_vec, rowvs[gi], cl])
                            accs[gi] = accs[gi] + v * qv
                            nacs[gi] = nacs[gi] + v * v
                    return tuple(accs), tuple(nacs)

                z = tuple(jnp.zeros((16,), jnp.float32) for _ in range(8))
                accs, nacs = lax.fori_loop(0, EMB // 2, colpair, (z, z))
                for gi in range(8):
                    key = accs[gi] * jnp.abs(accs[gi]) / nacs[gi]
                    key = jnp.where(valids[gi], key, _NEG)
                    cv, ci = plsc.sort_key_val(key, ivs[gi],
                                               descending=True)
                    tv, ti = _merge_top16(tv, ti, cv, ci)
            return tv, ti

        tv = jnp.full((16,), _NEG, jnp.float32)
        ti = jnp.zeros((16,), jnp.int32)
        dma_start(0, 0)
        dma_start(1, 1)

        def pair(c2, c):
            tv, ti = c
            ch0 = 2 * c2
            dma_wait(0)
            tv, ti = compute_chunk(ch0, 0, tv, ti)
            dma_start(ch0 + 2, 0)

            dma_wait(1)
            tv, ti = compute_chunk(ch0 + 1, 1, tv, ti)

            @pl.when(ch0 + 3 < _NCHUNK)
            def _():
                dma_start(ch0 + 3, 1)

            return tv, ti

        tv, ti = lax.fori_loop(0, (_NCHUNK - 1) // 2, pair, (tv, ti))
        dma_wait(0)
        tv, ti = compute_chunk(jnp.int32(_NCHUNK - 1), 0, tv, ti)

        tvbuf[...] = tv
        tibuf[...] = ti
        pltpu.sync_copy(tvbuf, keys_out.at[pl.ds(wid * 16, 16)])
        pltpu.sync_copy(tibuf, rows_out.at[pl.ds(wid * 16, 16)])

    return scan_k(base_emb, g_vec)


def _pick_call(base_emb, keys, rows):
    mesh = plsc.VectorSubcoreMesh(core_axis_name="c", subcore_axis_name="s",
                                  num_cores=_NC, num_subcores=_NS)

    @functools.partial(
        pl.kernel,
        out_type=(jax.ShapeDtypeStruct((EMB,), jnp.float32),
                  jax.ShapeDtypeStruct((16,), jnp.int32)),
        mesh=mesh,
        scratch_types=[
            pltpu.VMEM((_NW * 16,), jnp.float32),
            pltpu.VMEM((_NW * 16,), jnp.int32),
            pltpu.VMEM((16,), jnp.int32),
            pltpu.VMEM((16, EMB), jnp.float32),
            pltpu.VMEM((EMB,), jnp.float32),
            pltpu.SemaphoreType.DMA,
        ],
        compiler_params=pltpu.CompilerParams(needs_layout_passes=False),
    )
    def pick_k(emb_hbm, keys_hbm, rows_hbm, re_out, top8_out,
               kb, ib, tib, eb, oe, sem):
        cid = lax.axis_index("c")
        sid = lax.axis_index("s")

        @pl.when((cid == 0) & (sid == 0))
        def _():
            pltpu.sync_copy(keys_hbm, kb)
            pltpu.sync_copy(rows_hbm, ib)

            def mrg(w, c):
                tv, ti = c
                tv, ti = _merge_top16(tv, ti, kb[pl.ds(w * 16, 16)],
                                      ib[pl.ds(w * 16, 16)])
                return tv, ti

            tv = jnp.full((16,), _NEG, jnp.float32)
            ti = jnp.zeros((16,), jnp.int32)
            tv, ti = lax.fori_loop(0, _NW, mrg, (tv, ti))

            tib[...] = ti
            pltpu.sync_copy(tib, top8_out)
            pltpu.async_copy(emb_hbm.at[tib], eb, sem).wait()

            for k in range(8):
                s = jnp.zeros((16,), jnp.float32)
                for p in range(TOPK):
                    s = s + eb[p, pl.ds(16 * k, 16)]
                oe[pl.ds(16 * k, 16)] = s
            pltpu.sync_copy(oe, re_out)

    return pick_k(base_emb, keys, rows)


def _decode_body(idx_ref, lab_ref, q_ref, re_ref, w1_ref, b1_ref, w2_ref,
                 b2_ref, o_ref, acc_scr):
    i = pl.program_id(0)

    @pl.when(i == 0)
    def _():
        acc_scr[...] = jnp.zeros_like(acc_scr)

    col = idx_ref[i]
    b = col // 128
    c_in = col - b * 128
    tile = lab_ref[...]
    li = lax.broadcasted_iota(jnp.int32, (NUM_CLASS, 128), 1)
    tile = jnp.where(li < BASE_ROWS - b * 128, tile, 0.0)
    sel = (lax.broadcasted_iota(jnp.int32, (1, 128), 1)
           == c_in).astype(jnp.float32)
    acc_scr[...] += lax.dot_general(sel, tile, (((1,), (1,)), ((), ())),
                                    preferred_element_type=jnp.float32)

    @pl.when(i == TOPK - 1)
    def _():
        rag_label = acc_scr[...] * (1.0 / TOPK)
        hidden = (q_ref[...] * (1.0 - RETRIEVE_W)
                  + re_ref[...] * RETRIEVE_W)
        h1 = jnp.dot(hidden, w1_ref[...],
                     preferred_element_type=jnp.float32) + b1_ref[...]
        h1 = jnp.maximum(h1, 0.0)
        logits = jnp.dot(h1, w2_ref[...],
                         preferred_element_type=jnp.float32) + b2_ref[...]
        m = jnp.max(logits, axis=1, keepdims=True)
        e = jnp.exp(logits - m)
        sm = e / jnp.sum(e, axis=1, keepdims=True)
        o_ref[...] = sm * (1.0 - LABEL_W) + rag_label * LABEL_W


def _decode_call(top8, labels3d, q, rag_e, W1, b1, W2, b2, interpret=False):
    grid_spec = pltpu.PrefetchScalarGridSpec(
        num_scalar_prefetch=1,
        grid=(TOPK,),
        in_specs=[
            pl.BlockSpec((NUM_CLASS, 128), lambda i, idx: (0, idx[i] // 128)),
            pl.BlockSpec((1, EMB), lambda i, idx: (0, 0)),
            pl.BlockSpec((1, EMB), lambda i, idx: (0, 0)),
            pl.BlockSpec((EMB, EMB), lambda i, idx: (0, 0)),
            pl.BlockSpec((1, EMB), lambda i, idx: (0, 0)),
            pl.BlockSpec((EMB, NUM_CLASS), lambda i, idx: (0, 0)),
            pl.BlockSpec((1, NUM_CLASS), lambda i, idx: (0, 0)),
        ],
        out_specs=pl.BlockSpec((1, NUM_CLASS), lambda i, idx: (0, 0)),
        scratch_shapes=[pltpu.VMEM((1, NUM_CLASS), jnp.float32)],
    )
    return pl.pallas_call(
        _decode_body,
        grid_spec=grid_spec,
        out_shape=jax.ShapeDtypeStruct((1, NUM_CLASS), jnp.float32),
        interpret=interpret,
    )(top8, labels3d, q, rag_e, W1, b1, W2, b2)


def kernel(features, adj, W_pre, base_emb, base_labels, W1, b1, W2, b2):
    g2d, q2d = _gcn_call(features, adj, W_pre)
    keys, rows = _scan_call(base_emb, g2d.reshape((EMB,)))
    rag_e, top8 = _pick_call(base_emb, keys, rows)
    return _decode_call(top8, base_labels.T,
                        q2d, rag_e.reshape((1, EMB)),
                        W1, b1.reshape((1, EMB)), W2,
                        b2.reshape((1, NUM_CLASS)))


# speedup vs baseline: 1.0836x; 1.0251x over previous
import functools

import jax
import jax.numpy as jnp
from jax import lax
from jax.experimental import pallas as pl
from jax.experimental.pallas import tpu as pltpu
from jax.experimental.pallas import tpu_sc as plsc

N = 4096
D_FEAT = 256
EMB = 128
NUM_CLASS = 40
BASE_ROWS = 100000
TOPK = 8
RETRIEVE_W = 0.3
LABEL_W = 0.3

_RB = 512
_NB = N // _RB

_NC, _NS = 2, 16
_NW = _NC * _NS
_CH = 128
_NFULL = BASE_ROWS // _CH
_NCHUNK = 25
_NEG = -3.0e38


def _gcn_body(f_ref, w_ref, a_ref, g_ref, q_ref, h_scr, p_scr, cs_scr):
    i = pl.program_id(0)

    @pl.when(i == 0)
    def _():
        h_scr[...] = jnp.dot(f_ref[...], w_ref[...],
                             preferred_element_type=jnp.float32)
        cs_scr[...] = jnp.zeros_like(cs_scr)

    ablk = a_ref[...]
    p_scr[pl.ds(i * _RB, _RB), :] = jnp.tanh(
        jnp.dot(ablk, h_scr[...], preferred_element_type=jnp.float32))
    cs_scr[...] += jnp.sum(ablk, axis=0, keepdims=True)

    @pl.when(i == _NB - 1)
    def _():
        p_all = p_scr[...]
        g_ref[...] = jnp.sum(p_all, axis=0, keepdims=True) * (1.0 / N)
        q_ref[...] = jnp.dot(cs_scr[...] * (1.0 / N), p_all,
                             preferred_element_type=jnp.float32)


def _gcn_call(features, adj, W_pre, interpret=False):
    return pl.pallas_call(
        _gcn_body,
        grid=(_NB,),
        in_specs=[
            pl.BlockSpec((N, D_FEAT), lambda i: (0, 0)),
            pl.BlockSpec((D_FEAT, EMB), lambda i: (0, 0)),
            pl.BlockSpec((_RB, N), lambda i: (i, 0)),
        ],
        out_specs=[
            pl.BlockSpec((1, EMB), lambda i: (0, 0)),
            pl.BlockSpec((1, EMB), lambda i: (0, 0)),
        ],
        out_shape=[jax.ShapeDtypeStruct((1, EMB), jnp.float32)] * 2,
        scratch_shapes=[
            pltpu.VMEM((N, EMB), jnp.float32),
            pltpu.VMEM((N, EMB), jnp.float32),
            pltpu.VMEM((1, N), jnp.float32),
        ],
        interpret=interpret,
    )(features, W_pre, adj)


def _merge_top16(tv, ti, cv, ci):
    cvr = lax.rev(cv, (0,))
    cir = lax.rev(ci, (0,))
    keep = tv >= cvr
    mv = jnp.where(keep, tv, cvr)
    mi = jnp.where(keep, ti, cir)
    rv, ri = plsc.sort_key_val(mv, mi, descending=True)
    return rv, ri


def _scan_call(base_emb, g_vec):
    mesh = plsc.VectorSubcoreMesh(core_axis_name="c", subcore_axis_name="s",
                                  num_cores=_NC, num_subcores=_NS)

    @functools.partial(
        pl.kernel,
        out_type=(jax.ShapeDtypeStruct((_NW * 16,), jnp.float32),
                  jax.ShapeDtypeStruct((_NW * 16,), jnp.int32)),
        mesh=mesh,
        scratch_types=[
            pltpu.VMEM((2, _CH, EMB), jnp.float32),
            pltpu.VMEM((EMB,), jnp.float32),
            pltpu.VMEM((EMB * 16,), jnp.float32),
            pltpu.VMEM((16,), jnp.float32),
            pltpu.VMEM((16,), jnp.int32),
            pltpu.SemaphoreType.DMA,
            pltpu.SemaphoreType.DMA,
        ],
        compiler_params=pltpu.CompilerParams(needs_layout_passes=False),
    )
    def scan_k(emb_hbm, g_hbm, keys_out, rows_out,
               ebuf, qbuf, qrot, tvbuf, tibuf, sem0, sem1):
        cid = lax.axis_index("c")
        sid = lax.axis_index("s")
        wid = sid * _NC + cid
        lanes = lax.iota(jnp.int32, 16)

        def chunk_row0(t):
            c_eff = jnp.minimum(wid + _NW * t, _NFULL)
            row0 = jnp.minimum(c_eff * _CH, BASE_ROWS - _CH)
            return pl.multiple_of(row0, 8)

        pltpu.sync_copy(g_hbm, qbuf)

        def build_qrot(c, carry):
            qv = plsc.load_gather(qbuf,
                                  [jnp.bitwise_and(c + lanes, EMB - 1)])
            qrot[pl.ds(c * 16, 16)] = qv
            return carry

        lax.fori_loop(0, EMB, build_qrot, 0)

        sems = (sem0, sem1)

        def dma_start(t, slot):
            pltpu.async_copy(emb_hbm.at[pl.ds(chunk_row0(t), _CH)],
                             ebuf.at[slot], sems[slot])

        def dma_wait(slot):
            pltpu.make_async_copy(emb_hbm.at[pl.ds(0, _CH)],
                                  ebuf.at[slot], sems[slot]).wait()

        def compute_chunk(t, slot, tv, ti):
            c = wid + _NW * t
            in_range = c <= _NFULL
            c_eff = jnp.minimum(c, _NFULL)
            row0 = chunk_row0(t)
            slot_vec = jnp.full((16,), slot, jnp.int32)
            for half in range(_CH // 128):
                rowvs, valids, ivs = [], [], []
                for gi in range(8):
                    roff = half * 128 + gi * 16 + lanes
                    grow = row0 + roff
                    validv = jnp.logical_and(
                        jnp.logical_and(grow >= c_eff * _CH,
                                        grow < BASE_ROWS),
                        in_range)
                    rowvs.append(roff)
                    valids.append(validv)
                    ivs.append(grow)

                def colpair(cb, carry):
                    accs = list(carry[0])
                    nacs = list(carry[1])
                    for j in range(2):
                        cc = cb * 2 + j
                        cl = jnp.bitwise_and(cc + lanes, EMB - 1)
                        qv = qrot[pl.ds(cc * 16, 16)]
                        for gi in range(8):
                            v = plsc.load_gather(ebuf,
                                                 [slot_vec, rowvs[gi], cl])
                            accs[gi] = accs[gi] + v * qv
                            nacs[gi] = nacs[gi] + v * v
                    return tuple(accs), tuple(nacs)

                z = tuple(jnp.zeros((16,), jnp.float32) for _ in range(8))
                accs, nacs = lax.fori_loop(0, EMB // 2, colpair, (z, z))
                for gi in range(8):
                    key = accs[gi] * jnp.abs(accs[gi]) / nacs[gi]
                    key = jnp.where(valids[gi], key, _NEG)
                    cv, ci = plsc.sort_key_val(key, ivs[gi],
                                               descending=True)
                    tv, ti = _merge_top16(tv, ti, cv, ci)
            return tv, ti

        tv = jnp.full((16,), _NEG, jnp.float32)
        ti = jnp.zeros((16,), jnp.int32)
        dma_start(0, 0)
        dma_start(1, 1)

        def pair(c2, c):
            tv, ti = c
            ch0 = 2 * c2
            dma_wait(0)
            tv, ti = compute_chunk(ch0, 0, tv, ti)
            dma_start(ch0 + 2, 0)

            dma_wait(1)
            tv, ti = compute_chunk(ch0 + 1, 1, tv, ti)

            @pl.when(ch0 + 3 < _NCHUNK)
            def _():
                dma_start(ch0 + 3, 1)

            return tv, ti

        tv, ti = lax.fori_loop(0, (_NCHUNK - 1) // 2, pair, (tv, ti))
        dma_wait(0)
        tv, ti = compute_chunk(jnp.int32(_NCHUNK - 1), 0, tv, ti)

        tvbuf[...] = tv
        tibuf[...] = ti
        pltpu.sync_copy(tvbuf, keys_out.at[pl.ds(wid * 16, 16)])
        pltpu.sync_copy(tibuf, rows_out.at[pl.ds(wid * 16, 16)])

    return scan_k(base_emb, g_vec)


def _pick_call(base_emb, keys, rows):
    mesh = plsc.VectorSubcoreMesh(core_axis_name="c", subcore_axis_name="s",
                                  num_cores=_NC, num_subcores=_NS)

    @functools.partial(
        pl.kernel,
        out_type=(jax.ShapeDtypeStruct((EMB,), jnp.float32),
                  jax.ShapeDtypeStruct((16,), jnp.int32)),
        mesh=mesh,
        scratch_types=[
            pltpu.VMEM((_NW * 16,), jnp.float32),
            pltpu.VMEM((_NW * 16,), jnp.int32),
            pltpu.VMEM((16,), jnp.int32),
            pltpu.VMEM((16, EMB), jnp.float32),
            pltpu.VMEM((EMB,), jnp.float32),
            pltpu.SemaphoreType.DMA,
        ],
        compiler_params=pltpu.CompilerParams(needs_layout_passes=False),
    )
    def pick_k(emb_hbm, keys_hbm, rows_hbm, re_out, top8_out,
               kb, ib, tib, eb, oe, sem):
        cid = lax.axis_index("c")
        sid = lax.axis_index("s")

        @pl.when((cid == 0) & (sid == 0))
        def _():
            pltpu.sync_copy(keys_hbm, kb)
            pltpu.sync_copy(rows_hbm, ib)

            def mrg(w, c):
                tv, ti = c
                tv, ti = _merge_top16(tv, ti, kb[pl.ds(w * 16, 16)],
                                      ib[pl.ds(w * 16, 16)])
                return tv, ti

            tv = jnp.full((16,), _NEG, jnp.float32)
            ti = jnp.zeros((16,), jnp.int32)
            tv, ti = lax.fori_loop(0, _NW, mrg, (tv, ti))

            tib[...] = ti
            pltpu.sync_copy(tib, top8_out)
            pltpu.async_copy(emb_hbm.at[tib], eb, sem).wait()

            for k in range(8):
                s = jnp.zeros((16,), jnp.float32)
                for p in range(TOPK):
                    s = s + eb[p, pl.ds(16 * k, 16)]
                oe[pl.ds(16 * k, 16)] = s
            pltpu.sync_copy(oe, re_out)

    return pick_k(base_emb, keys, rows)


def _decode_body(idx_ref, lab_ref, q_ref, re_ref, w1_ref, b1_ref, w2_ref,
                 b2_ref, o_ref, acc_scr):
    i = pl.program_id(0)

    @pl.when(i == 0)
    def _():
        acc_scr[...] = jnp.zeros_like(acc_scr)

    col = idx_ref[i]
    b = col // 128
    c_in = col - b * 128
    tile = lab_ref[...]
    li = lax.broadcasted_iota(jnp.int32, (NUM_CLASS, 128), 1)
    tile = jnp.where(li < BASE_ROWS - b * 128, tile, 0.0)
    sel = (lax.broadcasted_iota(jnp.int32, (1, 128), 1)
           == c_in).astype(jnp.float32)
    acc_scr[...] += lax.dot_general(sel, tile, (((1,), (1,)), ((), ())),
                                    preferred_element_type=jnp.float32)

    @pl.when(i == TOPK - 1)
    def _():
        rag_label = acc_scr[...] * (1.0 / TOPK)
        hidden = (q_ref[...] * (1.0 - RETRIEVE_W)
                  + re_ref[...] * RETRIEVE_W)
        h1 = jnp.dot(hidden, w1_ref[...],
                     preferred_element_type=jnp.float32) + b1_ref[...]
        h1 = jnp.maximum(h1, 0.0)
        logits = jnp.dot(h1, w2_ref[...],
                         preferred_element_type=jnp.float32) + b2_ref[...]
        m = jnp.max(logits, axis=1, keepdims=True)
        e = jnp.exp(logits - m)
        sm = e / jnp.sum(e, axis=1, keepdims=True)
        o_ref[...] = sm * (1.0 - LABEL_W) + rag_label * LABEL_W


def _decode_call(top8, labels3d, q, rag_e, W1, b1, W2, b2, interpret=False):
    grid_spec = pltpu.PrefetchScalarGridSpec(
        num_scalar_prefetch=1,
        grid=(TOPK,),
        in_specs=[
            pl.BlockSpec((NUM_CLASS, 128), lambda i, idx: (0, idx[i] // 128)),
            pl.BlockSpec((1, EMB), lambda i, idx: (0, 0)),
            pl.BlockSpec((1, EMB), lambda i, idx: (0, 0)),
            pl.BlockSpec((EMB, EMB), lambda i, idx: (0, 0)),
            pl.BlockSpec((1, EMB), lambda i, idx: (0, 0)),
            pl.BlockSpec((EMB, NUM_CLASS), lambda i, idx: (0, 0)),
            pl.BlockSpec((1, NUM_CLASS), lambda i, idx: (0, 0)),
        ],
        out_specs=pl.BlockSpec((1, NUM_CLASS), lambda i, idx: (0, 0)),
        scratch_shapes=[pltpu.VMEM((1, NUM_CLASS), jnp.float32)],
    )
    return pl.pallas_call(
        _decode_body,
        grid_spec=grid_spec,
        out_shape=jax.ShapeDtypeStruct((1, NUM_CLASS), jnp.float32),
        interpret=interpret,
    )(top8, labels3d, q, rag_e, W1, b1, W2, b2)


def kernel(features, adj, W_pre, base_emb, base_labels, W1, b1, W2, b2):
    g2d, q2d = _gcn_call(features, adj, W_pre)
    keys, rows = _scan_call(base_emb, g2d.reshape((EMB,)))
    rag_e, top8 = _pick_call(base_emb, keys, rows)
    return _decode_call(top8, base_labels.T,
                        q2d, rag_e.reshape((1, EMB)),
                        W1, b1.reshape((1, EMB)), W2,
                        b2.reshape((1, NUM_CLASS)))
